# Initial kernel scaffold; baseline (speedup 1.0000x reference)
import jax
import jax.numpy as jnp
from jax.experimental import pallas as pl


def kernel(x, edge_index, W, b, Wc, bc):
    raise NotImplementedError("write your pallas kernel here")


# speedup vs baseline: 18.4408x; 18.4408x over previous
import functools

import jax
import jax.numpy as jnp
import numpy as np
from jax import lax
from jax.experimental import pallas as pl
from jax.experimental.pallas import tpu as pltpu
from jax.experimental.pallas import tpu_sc as plsc

N = 10000
E = 160000
D = 256

NUM_TILES = 16
N_PAD = 10240
E_PAD = 163840
NPT = N_PAD // NUM_TILES
EROWS = (E_PAD // 128) // NUM_TILES

_PAD_IDX = np.asarray(N + (np.arange(E_PAD - E) % (N_PAD - N)), dtype=np.int32)


def _tc_body(x_ref, w_ref, wc_ref, b_ref, bc_ref, t_ref, c_ref):
    w = jnp.dot(w_ref[...], wc_ref[...], preferred_element_type=jnp.float32)
    t_ref[...] = jnp.dot(x_ref[...], w, preferred_element_type=jnp.float32)
    c = jnp.dot(b_ref[...].reshape(1, D), wc_ref[...],
                preferred_element_type=jnp.float32)
    c_ref[...] = c + bc_ref[...].reshape(1, 1)


def _tc_call(x, W, Wc, b, bc):
    return pl.pallas_call(
        _tc_body,
        out_shape=[
            jax.ShapeDtypeStruct((N, 1), jnp.float32),
            jax.ShapeDtypeStruct((1, 1), jnp.float32),
        ],
    )(x, W, Wc, b, bc)


def _rsqrt16(x):
    i = plsc.bitcast(x, jnp.int32)
    i = jnp.int32(0x5F3759DF) - (i >> 1)
    y = plsc.bitcast(i, jnp.float32)
    for _ in range(3):
        y = y * (jnp.float32(1.5) - jnp.float32(0.5) * x * y * y)
    return y


def _sc_body(src_hbm, dst_hbm, t_hbm, c_hbm, ones_hbm, zeros_hbm, y_hbm,
             deg_s_sh, deg_d_sh, s_sh, agg_sh,
             src_v, dst_v, ones_v, vals_v,
             buf_a, buf_b, buf_c, buf_d, c_v):
    wid = lax.axis_index("s")
    ebase = wid * EROWS
    nbase = wid * NPT

    pltpu.sync_copy(zeros_hbm, deg_s_sh.at[pl.ds(nbase, NPT)])
    pltpu.sync_copy(zeros_hbm, deg_d_sh.at[pl.ds(nbase, NPT)])
    pltpu.sync_copy(zeros_hbm, agg_sh.at[pl.ds(nbase, NPT)])
    pltpu.sync_copy(src_hbm.at[pl.ds(ebase, EROWS)], src_v)
    pltpu.sync_copy(dst_hbm.at[pl.ds(ebase, EROWS)], dst_v)
    pltpu.sync_copy(ones_hbm, ones_v)
    pltpu.sync_copy(c_hbm, c_v)
    pltpu.sync_copy(t_hbm.at[pl.ds(nbase, NPT)], buf_b)
    plsc.subcore_barrier()

    for j in range(EROWS):
        pltpu.sync_copy(ones_v.at[j], deg_s_sh.at[src_v.at[j]], add=True)
        pltpu.sync_copy(ones_v.at[j], deg_d_sh.at[dst_v.at[j]], add=True)
    plsc.subcore_barrier()

    pltpu.sync_copy(deg_s_sh.at[pl.ds(nbase, NPT)], buf_a)
    for i in range(NPT // 16):
        d = jnp.maximum(buf_a[pl.ds(i * 16, 16)], jnp.float32(1.0))
        buf_c[pl.ds(i * 16, 16)] = buf_b[pl.ds(i * 16, 16)] * _rsqrt16(d)
    pltpu.sync_copy(buf_c, s_sh.at[pl.ds(nbase, NPT)])
    pltpu.sync_copy(deg_d_sh.at[pl.ds(nbase, NPT)], buf_a)
    for i in range(NPT // 16):
        d = jnp.maximum(buf_a[pl.ds(i * 16, 16)], jnp.float32(1.0))
        buf_d[pl.ds(i * 16, 16)] = _rsqrt16(d)
    plsc.subcore_barrier()

    for j in range(EROWS):
        pltpu.sync_copy(s_sh.at[src_v.at[j]], vals_v.at[j])
        pltpu.sync_copy(vals_v.at[j], agg_sh.at[dst_v.at[j]], add=True)
    plsc.subcore_barrier()

    pltpu.sync_copy(agg_sh.at[pl.ds(nbase, NPT)], buf_a)
    cvec = c_v[...]
    for i in range(NPT // 16):
        buf_c[pl.ds(i * 16, 16)] = buf_a[pl.ds(i * 16, 16)] * buf_d[pl.ds(i * 16, 16)] + cvec
    pltpu.sync_copy(buf_c, y_hbm.at[pl.ds(nbase, NPT)])


_sc_mesh = plsc.VectorSubcoreMesh(core_axis_name="c", subcore_axis_name="s",
                                  num_cores=1)

_sc_call = pl.kernel(
    _sc_body,
    out_type=jax.ShapeDtypeStruct((N_PAD,), jnp.float32),
    mesh=_sc_mesh,
    compiler_params=pltpu.CompilerParams(needs_layout_passes=False),
    scratch_types=[
        pltpu.VMEM_SHARED((N_PAD,), jnp.float32),
        pltpu.VMEM_SHARED((N_PAD,), jnp.float32),
        pltpu.VMEM_SHARED((N_PAD,), jnp.float32),
        pltpu.VMEM_SHARED((N_PAD,), jnp.float32),
        pltpu.VMEM((EROWS, 128), jnp.int32),
        pltpu.VMEM((EROWS, 128), jnp.int32),
        pltpu.VMEM((EROWS, 128), jnp.float32),
        pltpu.VMEM((EROWS, 128), jnp.float32),
        pltpu.VMEM((NPT,), jnp.float32),
        pltpu.VMEM((NPT,), jnp.float32),
        pltpu.VMEM((NPT,), jnp.float32),
        pltpu.VMEM((NPT,), jnp.float32),
        pltpu.VMEM((16,), jnp.float32),
    ],
)


@jax.jit
def kernel(x, edge_index, W, b, Wc, bc):
    t2d, c2d = _tc_call(x, W, Wc, b, bc)
    t_pad = jnp.concatenate([t2d[:, 0], jnp.zeros((N_PAD - N,), jnp.float32)])
    c16 = jnp.broadcast_to(c2d.reshape(()), (16,))
    pad = jnp.asarray(_PAD_IDX)
    src = jnp.concatenate([edge_index[0], pad]).reshape(E_PAD // 128, 128)
    dst = jnp.concatenate([edge_index[1], pad]).reshape(E_PAD // 128, 128)
    ones2d = jnp.ones((EROWS, 128), jnp.float32)
    zeros1 = jnp.zeros((NPT,), jnp.float32)
    y_pad = _sc_call(src, dst, t_pad, c16, ones2d, zeros1)
    return y_pad[:N].reshape(N, 1)


# speedup vs baseline: 26.4662x; 1.4352x over previous
import functools

import jax
import jax.numpy as jnp
import numpy as np
from jax import lax
from jax.experimental import pallas as pl
from jax.experimental.pallas import tpu as pltpu
from jax.experimental.pallas import tpu_sc as plsc

N = 10000
E = 160000
D = 256

NUM_TILES = 16
N_PAD = 10240
E_PAD = 163840
NPT = N_PAD // NUM_TILES
EROWS = (E_PAD // 128) // NUM_TILES

_PAD_IDX = np.asarray(N + (np.arange(E_PAD - E) % (N_PAD - N)), dtype=np.int32)


def _tc_body(x_ref, w_ref, wc_ref, b_ref, bc_ref, t_ref, c_ref):
    w = jnp.dot(w_ref[...], wc_ref[...], preferred_element_type=jnp.float32)
    t_ref[...] = jnp.dot(x_ref[...], w, preferred_element_type=jnp.float32)
    c = jnp.dot(b_ref[...].reshape(1, D), wc_ref[...],
                preferred_element_type=jnp.float32)
    c_ref[...] = c + bc_ref[...].reshape(1, 1)


def _tc_call(x, W, Wc, b, bc):
    return pl.pallas_call(
        _tc_body,
        out_shape=[
            jax.ShapeDtypeStruct((N, 1), jnp.float32),
            jax.ShapeDtypeStruct((1, 1), jnp.float32),
        ],
    )(x, W, Wc, b, bc)


def _rsqrt16(x):
    i = plsc.bitcast(x, jnp.int32)
    i = jnp.int32(0x5F3759DF) - (i >> 1)
    y = plsc.bitcast(i, jnp.float32)
    for _ in range(3):
        y = y * (jnp.float32(1.5) - jnp.float32(0.5) * x * y * y)
    return y


_K = 16


def _sc_body(src_hbm, dst_hbm, t_hbm, c_hbm, ones_hbm, zeros_hbm, y_hbm,
             deg_s_sh, deg_d_sh, s_sh, agg_sh,
             src_v, dst_v, ones_v, vals_v,
             buf_a, buf_b, buf_c, buf_d, c_v,
             sem_in, sem_g, sem_s):
    wid = lax.axis_index("s")
    ebase = wid * EROWS
    nbase = wid * NPT

    p0 = [
        pltpu.async_copy(zeros_hbm, deg_s_sh.at[pl.ds(nbase, NPT)], sem_in),
        pltpu.async_copy(zeros_hbm, deg_d_sh.at[pl.ds(nbase, NPT)], sem_in),
        pltpu.async_copy(zeros_hbm, agg_sh.at[pl.ds(nbase, NPT)], sem_in),
        pltpu.async_copy(src_hbm.at[pl.ds(ebase, EROWS)], src_v, sem_in),
        pltpu.async_copy(dst_hbm.at[pl.ds(ebase, EROWS)], dst_v, sem_in),
        pltpu.async_copy(ones_hbm, ones_v, sem_in),
        pltpu.async_copy(c_hbm, c_v, sem_in),
        pltpu.async_copy(t_hbm.at[pl.ds(nbase, NPT)], buf_b, sem_in),
    ]
    for d in p0:
        d.wait()
    plsc.subcore_barrier()

    for g in range(0, EROWS, _K):
        descs = []
        for j in range(g, g + _K):
            descs.append(pltpu.async_copy(
                ones_v, deg_s_sh.at[src_v.at[j]], sem_s, add=True))
            descs.append(pltpu.async_copy(
                ones_v, deg_d_sh.at[dst_v.at[j]], sem_s, add=True))
        for d in descs:
            d.wait()
    plsc.subcore_barrier()

    da = pltpu.async_copy(deg_s_sh.at[pl.ds(nbase, NPT)], buf_a, sem_in)
    dd = pltpu.async_copy(deg_d_sh.at[pl.ds(nbase, NPT)], buf_d, sem_in)
    da.wait()
    for i in range(NPT // 16):
        d = jnp.maximum(buf_a[pl.ds(i * 16, 16)], jnp.float32(1.0))
        buf_c[pl.ds(i * 16, 16)] = buf_b[pl.ds(i * 16, 16)] * _rsqrt16(d)
    ds = pltpu.async_copy(buf_c, s_sh.at[pl.ds(nbase, NPT)], sem_in)
    dd.wait()
    for i in range(NPT // 16):
        d = jnp.maximum(buf_d[pl.ds(i * 16, 16)], jnp.float32(1.0))
        buf_d[pl.ds(i * 16, 16)] = _rsqrt16(d)
    ds.wait()
    plsc.subcore_barrier()

    pend = []
    for g in range(0, EROWS, _K):
        gd = [pltpu.async_copy(s_sh.at[src_v.at[j]], vals_v.at[j], sem_g)
              for j in range(g, g + _K)]
        for d in gd:
            d.wait()
        for d in pend:
            d.wait()
        pend = [pltpu.async_copy(vals_v.at[j], agg_sh.at[dst_v.at[j]],
                                 sem_s, add=True)
                for j in range(g, g + _K)]
    for d in pend:
        d.wait()
    plsc.subcore_barrier()

    pltpu.sync_copy(agg_sh.at[pl.ds(nbase, NPT)], buf_a)
    cvec = c_v[...]
    for i in range(NPT // 16):
        buf_c[pl.ds(i * 16, 16)] = buf_a[pl.ds(i * 16, 16)] * buf_d[pl.ds(i * 16, 16)] + cvec
    pltpu.sync_copy(buf_c, y_hbm.at[pl.ds(nbase, NPT)])


_sc_mesh = plsc.VectorSubcoreMesh(core_axis_name="c", subcore_axis_name="s",
                                  num_cores=1)

_sc_call = pl.kernel(
    _sc_body,
    out_type=jax.ShapeDtypeStruct((N_PAD,), jnp.float32),
    mesh=_sc_mesh,
    compiler_params=pltpu.CompilerParams(needs_layout_passes=False),
    scratch_types=[
        pltpu.VMEM_SHARED((N_PAD,), jnp.float32),
        pltpu.VMEM_SHARED((N_PAD,), jnp.float32),
        pltpu.VMEM_SHARED((N_PAD,), jnp.float32),
        pltpu.VMEM_SHARED((N_PAD,), jnp.float32),
        pltpu.VMEM((EROWS, 128), jnp.int32),
        pltpu.VMEM((EROWS, 128), jnp.int32),
        pltpu.VMEM((128,), jnp.float32),
        pltpu.VMEM((EROWS, 128), jnp.float32),
        pltpu.VMEM((NPT,), jnp.float32),
        pltpu.VMEM((NPT,), jnp.float32),
        pltpu.VMEM((NPT,), jnp.float32),
        pltpu.VMEM((NPT,), jnp.float32),
        pltpu.VMEM((16,), jnp.float32),
        pltpu.SemaphoreType.DMA,
        pltpu.SemaphoreType.DMA,
        pltpu.SemaphoreType.DMA,
    ],
)


@jax.jit
def kernel(x, edge_index, W, b, Wc, bc):
    t2d, c2d = _tc_call(x, W, Wc, b, bc)
    t_pad = jnp.concatenate([t2d[:, 0], jnp.zeros((N_PAD - N,), jnp.float32)])
    c16 = jnp.broadcast_to(c2d.reshape(()), (16,))
    pad = jnp.asarray(_PAD_IDX)
    src = jnp.concatenate([edge_index[0], pad]).reshape(E_PAD // 128, 128)
    dst = jnp.concatenate([edge_index[1], pad]).reshape(E_PAD // 128, 128)
    ones2d = jnp.ones((128,), jnp.float32)
    zeros1 = jnp.zeros((NPT,), jnp.float32)
    y_pad = _sc_call(src, dst, t_pad, c16, ones2d, zeros1)
    return y_pad[:N].reshape(N, 1)


# speedup vs baseline: 28.5381x; 1.0783x over previous
import jax
import jax.numpy as jnp
import numpy as np
from jax import lax
from jax.experimental import pallas as pl
from jax.experimental.pallas import tpu as pltpu
from jax.experimental.pallas import tpu_sc as plsc

N = 10000
E = 160000
D = 256

NUM_TILES = 16
N_PAD = 10240
E_PAD = 163840
NPT = N_PAD // NUM_TILES
EROWS = (E_PAD // 128) // NUM_TILES
DROWS = E_PAD // 128

_PAD2 = np.broadcast_to(
    np.asarray(N + (np.arange(E_PAD - E) % (N_PAD - N)), dtype=np.int32),
    (2, E_PAD - E))


def _tc_body(x_ref, w_ref, wc_ref, b_ref, bc_ref, t_ref, c_ref):
    w = jnp.dot(w_ref[...], wc_ref[...], preferred_element_type=jnp.float32)
    t_ref[...] = jnp.dot(x_ref[...], w, preferred_element_type=jnp.float32)
    c = jnp.dot(b_ref[...].reshape(1, D), wc_ref[...],
                preferred_element_type=jnp.float32)
    c_ref[...] = c + bc_ref[...].reshape(1, 1)


def _tc_call(x, W, Wc, b, bc):
    return pl.pallas_call(
        _tc_body,
        out_shape=[
            jax.ShapeDtypeStruct((N, 1), jnp.float32),
            jax.ShapeDtypeStruct((1, 1), jnp.float32),
        ],
    )(x, W, Wc, b, bc)


def _rsqrt16(x):
    i = plsc.bitcast(x, jnp.int32)
    i = jnp.int32(0x5F3759DF) - (i >> 1)
    y = plsc.bitcast(i, jnp.float32)
    for _ in range(3):
        y = y * (jnp.float32(1.5) - jnp.float32(0.5) * x * y * y)
    return y


_K = 16


def _sc_body(ei_hbm, t_hbm, c_hbm, ones_hbm, zeros_hbm, y_hbm,
             deg_s_sh, deg_d_sh, s_sh, agg_sh,
             src_v, dst_v, ones_v, vals_v,
             buf_a, buf_b, buf_c, buf_d, c_v,
             sem_in, sem_g, sem_s):
    wid = lax.axis_index("s")
    ebase = wid * EROWS
    nbase = wid * NPT

    p0 = [
        pltpu.async_copy(zeros_hbm, deg_s_sh.at[pl.ds(nbase, NPT)], sem_in),
        pltpu.async_copy(zeros_hbm, deg_d_sh.at[pl.ds(nbase, NPT)], sem_in),
        pltpu.async_copy(zeros_hbm, agg_sh.at[pl.ds(nbase, NPT)], sem_in),
        pltpu.async_copy(ei_hbm.at[pl.ds(ebase, EROWS)], src_v, sem_in),
        pltpu.async_copy(ei_hbm.at[pl.ds(DROWS + ebase, EROWS)], dst_v, sem_in),
        pltpu.async_copy(ones_hbm, ones_v, sem_in),
        pltpu.async_copy(c_hbm, c_v, sem_in),
        pltpu.async_copy(t_hbm.at[pl.ds(nbase, NPT)], buf_b, sem_in),
    ]
    for d in p0:
        d.wait()
    plsc.subcore_barrier()

    for g in range(0, EROWS, _K):
        descs = []
        for j in range(g, g + _K):
            descs.append(pltpu.async_copy(
                ones_v, deg_s_sh.at[src_v.at[j]], sem_s, add=True))
            descs.append(pltpu.async_copy(
                ones_v, deg_d_sh.at[dst_v.at[j]], sem_s, add=True))
        for d in descs:
            d.wait()
    plsc.subcore_barrier()

    da = pltpu.async_copy(deg_s_sh.at[pl.ds(nbase, NPT)], buf_a, sem_in)
    dd = pltpu.async_copy(deg_d_sh.at[pl.ds(nbase, NPT)], buf_d, sem_in)
    da.wait()
    for i in range(NPT // 16):
        d = jnp.maximum(buf_a[pl.ds(i * 16, 16)], jnp.float32(1.0))
        buf_c[pl.ds(i * 16, 16)] = buf_b[pl.ds(i * 16, 16)] * _rsqrt16(d)
    ds = pltpu.async_copy(buf_c, s_sh.at[pl.ds(nbase, NPT)], sem_in)
    dd.wait()
    for i in range(NPT // 16):
        d = jnp.maximum(buf_d[pl.ds(i * 16, 16)], jnp.float32(1.0))
        buf_d[pl.ds(i * 16, 16)] = _rsqrt16(d)
    ds.wait()
    plsc.subcore_barrier()

    pend = []
    for g in range(0, EROWS, _K):
        gd = [pltpu.async_copy(s_sh.at[src_v.at[j]], vals_v.at[j], sem_g)
              for j in range(g, g + _K)]
        for d in gd:
            d.wait()
        for d in pend:
            d.wait()
        pend = [pltpu.async_copy(vals_v.at[j], agg_sh.at[dst_v.at[j]],
                                 sem_s, add=True)
                for j in range(g, g + _K)]
    for d in pend:
        d.wait()
    plsc.subcore_barrier()

    pltpu.sync_copy(agg_sh.at[pl.ds(nbase, NPT)], buf_a)
    cvec = c_v[...]
    for i in range(NPT // 16):
        buf_c[pl.ds(i * 16, 16)] = buf_a[pl.ds(i * 16, 16)] * buf_d[pl.ds(i * 16, 16)] + cvec
    pltpu.sync_copy(buf_c, y_hbm.at[pl.ds(nbase, NPT)])


_sc_mesh = plsc.VectorSubcoreMesh(core_axis_name="c", subcore_axis_name="s",
                                  num_cores=1)

_sc_call = pl.kernel(
    _sc_body,
    out_type=jax.ShapeDtypeStruct((N_PAD,), jnp.float32),
    mesh=_sc_mesh,
    compiler_params=pltpu.CompilerParams(needs_layout_passes=False),
    scratch_types=[
        pltpu.VMEM_SHARED((N_PAD,), jnp.float32),
        pltpu.VMEM_SHARED((N_PAD,), jnp.float32),
        pltpu.VMEM_SHARED((N_PAD,), jnp.float32),
        pltpu.VMEM_SHARED((N_PAD,), jnp.float32),
        pltpu.VMEM((EROWS, 128), jnp.int32),
        pltpu.VMEM((EROWS, 128), jnp.int32),
        pltpu.VMEM((128,), jnp.float32),
        pltpu.VMEM((EROWS, 128), jnp.float32),
        pltpu.VMEM((NPT,), jnp.float32),
        pltpu.VMEM((NPT,), jnp.float32),
        pltpu.VMEM((NPT,), jnp.float32),
        pltpu.VMEM((NPT,), jnp.float32),
        pltpu.VMEM((16,), jnp.float32),
        pltpu.SemaphoreType.DMA,
        pltpu.SemaphoreType.DMA,
        pltpu.SemaphoreType.DMA,
    ],
)


@jax.jit
def kernel(x, edge_index, W, b, Wc, bc):
    t2d, c2d = _tc_call(x, W, Wc, b, bc)
    t_pad = jnp.concatenate([t2d[:, 0], jnp.zeros((N_PAD - N,), jnp.float32)])
    c16 = jnp.broadcast_to(c2d.reshape(()), (16,))
    ei = jnp.concatenate([edge_index, jnp.asarray(_PAD2)], axis=1)
    ei = ei.reshape(2 * DROWS, 128)
    ones1 = jnp.ones((128,), jnp.float32)
    zeros1 = jnp.zeros((NPT,), jnp.float32)
    y_pad = _sc_call(ei, t_pad, c16, ones1, zeros1)
    return y_pad[:N].reshape(N, 1)


# speedup vs baseline: 32.9374x; 1.1542x over previous
import jax
import jax.numpy as jnp
from jax import lax
from jax.experimental import pallas as pl
from jax.experimental.pallas import tpu as pltpu
from jax.experimental.pallas import tpu_sc as plsc

import numpy as np

N = 10000
E = 160000
D = 256

NUM_TILES = 16
N_PAD = 10240
E_PAD = 163840
NPT = N_PAD // NUM_TILES
EROWS = (E_PAD // 128) // NUM_TILES
DROWS = E_PAD // 128

_PAD2 = np.broadcast_to(
    np.asarray(N + (np.arange(E_PAD - E) % (N_PAD - N)), dtype=np.int32),
    (2, E_PAD - E))


def _tc_body(x_ref, w_ref, wc_ref, b_ref, bc_ref, t_ref, c_ref):
    wT = lax.dot_general(wc_ref[...], w_ref[...], (((0,), (1,)), ((), ())),
                         preferred_element_type=jnp.float32)
    tT = lax.dot_general(wT, x_ref[...], (((1,), (1,)), ((), ())),
                         preferred_element_type=jnp.float32)
    t_ref[pl.ds(0, N)] = tT.reshape((N,))
    t_ref[pl.ds(N, N_PAD - N)] = jnp.zeros((N_PAD - N,), jnp.float32)
    c = jnp.dot(b_ref[...].reshape(1, D), wc_ref[...],
                preferred_element_type=jnp.float32)
    c_ref[...] = c + bc_ref[...].reshape(1, 1)


def _tc_call(x, W, Wc, b, bc):
    return pl.pallas_call(
        _tc_body,
        out_shape=[
            jax.ShapeDtypeStruct((N_PAD,), jnp.float32),
            jax.ShapeDtypeStruct((1, 1), jnp.float32),
        ],
    )(x, W, Wc, b, bc)


def _rsqrt16(x):
    i = plsc.bitcast(x, jnp.int32)
    i = jnp.int32(0x5F3759DF) - (i >> 1)
    y = plsc.bitcast(i, jnp.float32)
    for _ in range(3):
        y = y * (jnp.float32(1.5) - jnp.float32(0.5) * x * y * y)
    return y


_K = 16


def _sc_body(ei_hbm, t_hbm, c_hbm, ones_hbm, zeros_hbm, y_hbm,
             deg_s_sh, deg_d_sh, s_sh, agg_sh,
             src_v, dst_v, ones_v, vals_v,
             buf_t, buf_a, buf_c, buf_d, c_v,
             sem_in, sem_g, sem_s):
    wid = lax.axis_index("s")
    ebase = wid * EROWS
    nbase = wid * NPT

    p0 = [
        pltpu.async_copy(zeros_hbm, deg_s_sh.at[pl.ds(nbase, NPT)], sem_in),
        pltpu.async_copy(zeros_hbm, deg_d_sh.at[pl.ds(nbase, NPT)], sem_in),
        pltpu.async_copy(zeros_hbm, agg_sh.at[pl.ds(nbase, NPT)], sem_in),
        pltpu.async_copy(ei_hbm.at[pl.ds(ebase, EROWS)], src_v, sem_in),
        pltpu.async_copy(ei_hbm.at[pl.ds(DROWS + ebase, EROWS)], dst_v, sem_in),
        pltpu.async_copy(ones_hbm, ones_v, sem_in),
        pltpu.async_copy(c_hbm, c_v, sem_in),
        pltpu.async_copy(t_hbm.at[pl.ds(nbase, NPT)], buf_t, sem_in),
    ]
    for d in p0:
        d.wait()
    plsc.subcore_barrier()

    for g in range(0, EROWS, _K):
        descs = []
        for j in range(g, g + _K):
            descs.append(pltpu.async_copy(
                ones_v, deg_s_sh.at[src_v.at[j]], sem_s, add=True))
            descs.append(pltpu.async_copy(
                ones_v, deg_d_sh.at[dst_v.at[j]], sem_s, add=True))
        for d in descs:
            d.wait()
    plsc.subcore_barrier()

    da = pltpu.async_copy(deg_s_sh.at[pl.ds(nbase, NPT)], buf_a, sem_in)
    dd = pltpu.async_copy(deg_d_sh.at[pl.ds(nbase, NPT)], buf_d, sem_in)
    da.wait()
    for i in range(NPT // 16):
        d = jnp.maximum(buf_a[pl.ds(i * 16, 16)], jnp.float32(1.0))
        buf_c[pl.ds(i * 16, 16)] = buf_t[pl.ds(i * 16, 16)] * _rsqrt16(d)
    ds = pltpu.async_copy(buf_c, s_sh.at[pl.ds(nbase, NPT)], sem_in)
    dd.wait()
    for i in range(NPT // 16):
        d = jnp.maximum(buf_d[pl.ds(i * 16, 16)], jnp.float32(1.0))
        buf_d[pl.ds(i * 16, 16)] = _rsqrt16(d)
    ds.wait()
    plsc.subcore_barrier()

    pend = []
    for g in range(0, EROWS, _K):
        gd = [pltpu.async_copy(s_sh.at[src_v.at[j]], vals_v.at[j], sem_g)
              for j in range(g, g + _K)]
        for d in gd:
            d.wait()
        for d in pend:
            d.wait()
        pend = [pltpu.async_copy(vals_v.at[j], agg_sh.at[dst_v.at[j]],
                                 sem_s, add=True)
                for j in range(g, g + _K)]
    for d in pend:
        d.wait()
    plsc.subcore_barrier()

    pltpu.sync_copy(agg_sh.at[pl.ds(nbase, NPT)], buf_a)
    cvec = c_v[...]
    for i in range(NPT // 16):
        buf_c[pl.ds(i * 16, 16)] = buf_a[pl.ds(i * 16, 16)] * buf_d[pl.ds(i * 16, 16)] + cvec
    pltpu.sync_copy(buf_c, y_hbm.at[pl.ds(nbase, NPT)])


_sc_mesh = plsc.VectorSubcoreMesh(core_axis_name="c", subcore_axis_name="s",
                                  num_cores=1)

_sc_call = pl.kernel(
    _sc_body,
    out_type=jax.ShapeDtypeStruct((N_PAD,), jnp.float32),
    mesh=_sc_mesh,
    compiler_params=pltpu.CompilerParams(needs_layout_passes=False),
    scratch_types=[
        pltpu.VMEM_SHARED((N_PAD,), jnp.float32),
        pltpu.VMEM_SHARED((N_PAD,), jnp.float32),
        pltpu.VMEM_SHARED((N_PAD,), jnp.float32),
        pltpu.VMEM_SHARED((N_PAD,), jnp.float32),
        pltpu.VMEM((EROWS, 128), jnp.int32),
        pltpu.VMEM((EROWS, 128), jnp.int32),
        pltpu.VMEM((128,), jnp.float32),
        pltpu.VMEM((EROWS, 128), jnp.float32),
        pltpu.VMEM((NPT,), jnp.float32),
        pltpu.VMEM((NPT,), jnp.float32),
        pltpu.VMEM((NPT,), jnp.float32),
        pltpu.VMEM((NPT,), jnp.float32),
        pltpu.VMEM((16,), jnp.float32),
        pltpu.SemaphoreType.DMA,
        pltpu.SemaphoreType.DMA,
        pltpu.SemaphoreType.DMA,
    ],
)


@jax.jit
def kernel(x, edge_index, W, b, Wc, bc):
    t1d, c2d = _tc_call(x, W, Wc, b, bc)
    c16 = jnp.broadcast_to(c2d.reshape(()), (16,))
    ei = jnp.concatenate([edge_index, jnp.asarray(_PAD2)], axis=1)
    ei = ei.reshape(2 * DROWS, 128)
    ones1 = jnp.ones((128,), jnp.float32)
    zeros1 = jnp.zeros((NPT,), jnp.float32)
    y_pad = _sc_call(ei, t1d, c16, ones1, zeros1)
    return y_pad[:N].reshape(N, 1)


# speedup vs baseline: 33.7159x; 1.0236x over previous
import jax
import jax.numpy as jnp
from jax import lax
from jax.experimental import pallas as pl
from jax.experimental.pallas import tpu as pltpu
from jax.experimental.pallas import tpu_sc as plsc

N = 10000
E = 160000
D = 256

NUM_TILES = 16
N_PAD = 10240
NPT = N_PAD // NUM_TILES
EPT = E // NUM_TILES
FULL = EPT // 128
TAIL = EPT - FULL * 128


def _tc_body(x_ref, w_ref, wc_ref, b_ref, bc_ref, t_ref, c_ref):
    wT = lax.dot_general(wc_ref[...], w_ref[...], (((0,), (1,)), ((), ())),
                         preferred_element_type=jnp.float32)
    tT = lax.dot_general(wT, x_ref[...], (((1,), (1,)), ((), ())),
                         preferred_element_type=jnp.float32)
    t_ref[pl.ds(0, N)] = tT.reshape((N,))
    t_ref[pl.ds(N, N_PAD - N)] = jnp.zeros((N_PAD - N,), jnp.float32)
    c = jnp.dot(b_ref[...].reshape(1, D), wc_ref[...],
                preferred_element_type=jnp.float32)
    c_ref[...] = c + bc_ref[...].reshape(1, 1)


def _tc_call(x, W, Wc, b, bc):
    return pl.pallas_call(
        _tc_body,
        out_shape=[
            jax.ShapeDtypeStruct((N_PAD,), jnp.float32),
            jax.ShapeDtypeStruct((1, 1), jnp.float32),
        ],
    )(x, W, Wc, b, bc)


def _rsqrt16(x):
    i = plsc.bitcast(x, jnp.int32)
    i = jnp.int32(0x5F3759DF) - (i >> 1)
    y = plsc.bitcast(i, jnp.float32)
    for _ in range(3):
        y = y * (jnp.float32(1.5) - jnp.float32(0.5) * x * y * y)
    return y


_K = 16

_CHUNKS = [(j * 128, 128) for j in range(FULL)] + [(FULL * 128, TAIL)]


def _sc_body(ei_hbm, t_hbm, c_hbm, ones_hbm, zeros_hbm, y_hbm,
             deg_s_sh, deg_d_sh, s_sh, agg_sh,
             src_v, dst_v, ones_v, vals_v,
             buf_t, buf_a, buf_c, buf_d, c_v,
             sem_in, sem_g, sem_s):
    wid = lax.axis_index("s")
    ebase = wid * EPT
    nbase = wid * NPT

    p0 = [
        pltpu.async_copy(zeros_hbm, deg_s_sh.at[pl.ds(nbase, NPT)], sem_in),
        pltpu.async_copy(zeros_hbm, deg_d_sh.at[pl.ds(nbase, NPT)], sem_in),
        pltpu.async_copy(zeros_hbm, agg_sh.at[pl.ds(nbase, NPT)], sem_in),
        pltpu.async_copy(ei_hbm.at[pl.ds(ebase, EPT)],
                         src_v.at[pl.ds(0, EPT)], sem_in),
        pltpu.async_copy(ei_hbm.at[pl.ds(E + ebase, EPT)],
                         dst_v.at[pl.ds(0, EPT)], sem_in),
        pltpu.async_copy(ones_hbm, ones_v, sem_in),
        pltpu.async_copy(c_hbm, c_v, sem_in),
        pltpu.async_copy(t_hbm.at[pl.ds(nbase, NPT)], buf_t, sem_in),
    ]
    for d in p0:
        d.wait()
    plsc.subcore_barrier()

    for g in range(0, len(_CHUNKS), _K):
        descs = []
        for off, ln in _CHUNKS[g:g + _K]:
            descs.append(pltpu.async_copy(
                ones_v.at[pl.ds(0, ln)],
                deg_s_sh.at[src_v.at[pl.ds(off, ln)]], sem_s, add=True))
            descs.append(pltpu.async_copy(
                ones_v.at[pl.ds(0, ln)],
                deg_d_sh.at[dst_v.at[pl.ds(off, ln)]], sem_s, add=True))
        for d in descs:
            d.wait()
    plsc.subcore_barrier()

    da = pltpu.async_copy(deg_s_sh.at[pl.ds(nbase, NPT)], buf_a, sem_in)
    dd = pltpu.async_copy(deg_d_sh.at[pl.ds(nbase, NPT)], buf_d, sem_in)
    da.wait()
    for i in range(NPT // 16):
        d = jnp.maximum(buf_a[pl.ds(i * 16, 16)], jnp.float32(1.0))
        buf_c[pl.ds(i * 16, 16)] = buf_t[pl.ds(i * 16, 16)] * _rsqrt16(d)
    ds = pltpu.async_copy(buf_c, s_sh.at[pl.ds(nbase, NPT)], sem_in)
    dd.wait()
    for i in range(NPT // 16):
        d = jnp.maximum(buf_d[pl.ds(i * 16, 16)], jnp.float32(1.0))
        buf_d[pl.ds(i * 16, 16)] = _rsqrt16(d)
    ds.wait()
    plsc.subcore_barrier()

    pend = []
    for g in range(0, len(_CHUNKS), _K):
        gd = [pltpu.async_copy(s_sh.at[src_v.at[pl.ds(off, ln)]],
                               vals_v.at[pl.ds(off, ln)], sem_g)
              for off, ln in _CHUNKS[g:g + _K]]
        for d in gd:
            d.wait()
        for d in pend:
            d.wait()
        pend = [pltpu.async_copy(vals_v.at[pl.ds(off, ln)],
                                 agg_sh.at[dst_v.at[pl.ds(off, ln)]],
                                 sem_s, add=True)
                for off, ln in _CHUNKS[g:g + _K]]
    for d in pend:
        d.wait()
    plsc.subcore_barrier()

    pltpu.sync_copy(agg_sh.at[pl.ds(nbase, NPT)], buf_a)
    cvec = c_v[...]
    for i in range(NPT // 16):
        buf_c[pl.ds(i * 16, 16)] = buf_a[pl.ds(i * 16, 16)] * buf_d[pl.ds(i * 16, 16)] + cvec
    pltpu.sync_copy(buf_c, y_hbm.at[pl.ds(nbase, NPT)])


_sc_mesh = plsc.VectorSubcoreMesh(core_axis_name="c", subcore_axis_name="s",
                                  num_cores=1)

_sc_call = pl.kernel(
    _sc_body,
    out_type=jax.ShapeDtypeStruct((N_PAD,), jnp.float32),
    mesh=_sc_mesh,
    compiler_params=pltpu.CompilerParams(needs_layout_passes=False),
    scratch_types=[
        pltpu.VMEM_SHARED((N_PAD,), jnp.float32),
        pltpu.VMEM_SHARED((N_PAD,), jnp.float32),
        pltpu.VMEM_SHARED((N_PAD,), jnp.float32),
        pltpu.VMEM_SHARED((N_PAD,), jnp.float32),
        pltpu.VMEM((EPT,), jnp.int32),
        pltpu.VMEM((EPT,), jnp.int32),
        pltpu.VMEM((128,), jnp.float32),
        pltpu.VMEM((EPT,), jnp.float32),
        pltpu.VMEM((NPT,), jnp.float32),
        pltpu.VMEM((NPT,), jnp.float32),
        pltpu.VMEM((NPT,), jnp.float32),
        pltpu.VMEM((NPT,), jnp.float32),
        pltpu.VMEM((16,), jnp.float32),
        pltpu.SemaphoreType.DMA,
        pltpu.SemaphoreType.DMA,
        pltpu.SemaphoreType.DMA,
    ],
)


@jax.jit
def kernel(x, edge_index, W, b, Wc, bc):
    t1d, c2d = _tc_call(x, W, Wc, b, bc)
    c16 = jnp.broadcast_to(c2d.reshape(()), (16,))
    ei = jnp.reshape(edge_index, (2 * E,))
    ones1 = jnp.ones((128,), jnp.float32)
    zeros1 = jnp.zeros((NPT,), jnp.float32)
    y_pad = _sc_call(ei, t1d, c16, ones1, zeros1)
    return y_pad[:N].reshape(N, 1)


# speedup vs baseline: 36.2254x; 1.0744x over previous
import jax
import jax.numpy as jnp
from jax import lax
from jax.experimental import pallas as pl
from jax.experimental.pallas import tpu as pltpu
from jax.experimental.pallas import tpu_sc as plsc

N = 10000
E = 160000
D = 256

NUM_CORES = 2
NUM_TILES = 16
NUM_W = NUM_CORES * NUM_TILES
N_PAD = 10240
NPT = N_PAD // NUM_TILES
EPW = E // NUM_W
WFULL = EPW // 128
WTAIL = EPW - WFULL * 128

_WCHUNKS = [(j * 128, 128) for j in range(WFULL)] + [(WFULL * 128, WTAIL)]

_K = 32


def _tc_body(x_ref, w_ref, wc_ref, b_ref, bc_ref, t_ref, c_ref):
    wT = lax.dot_general(wc_ref[...], w_ref[...], (((0,), (1,)), ((), ())),
                         preferred_element_type=jnp.float32)
    tT = lax.dot_general(wT, x_ref[...], (((1,), (1,)), ((), ())),
                         preferred_element_type=jnp.float32)
    t_ref[pl.ds(0, N)] = tT.reshape((N,))
    t_ref[pl.ds(N, N_PAD - N)] = jnp.zeros((N_PAD - N,), jnp.float32)
    c = jnp.dot(b_ref[...].reshape(1, D), wc_ref[...],
                preferred_element_type=jnp.float32)
    c_ref[...] = c + bc_ref[...].reshape(1, 1)


def _tc_call(x, W, Wc, b, bc):
    return pl.pallas_call(
        _tc_body,
        out_shape=[
            jax.ShapeDtypeStruct((N_PAD,), jnp.float32),
            jax.ShapeDtypeStruct((1, 1), jnp.float32),
        ],
    )(x, W, Wc, b, bc)


def _tc2_body(deg_ref, agg_ref, c_ref, y_ref):
    dd = deg_ref[pl.ds(N_PAD, N_PAD)] + deg_ref[pl.ds(3 * N_PAD, N_PAD)]
    agg = agg_ref[pl.ds(0, N_PAD)] + agg_ref[pl.ds(N_PAD, N_PAD)]
    nd = lax.rsqrt(jnp.maximum(dd, jnp.float32(1.0)))
    y_ref[...] = agg * nd + c_ref[0, 0]


def _tc2_call(deg, agg, c2d):
    return pl.pallas_call(
        _tc2_body,
        out_shape=jax.ShapeDtypeStruct((N_PAD,), jnp.float32),
    )(deg, agg, c2d)


def _rsqrt16(x):
    i = plsc.bitcast(x, jnp.int32)
    i = jnp.int32(0x5F3759DF) - (i >> 1)
    y = plsc.bitcast(i, jnp.float32)
    for _ in range(3):
        y = y * (jnp.float32(1.5) - jnp.float32(0.5) * x * y * y)
    return y


def _sca_body(ei_hbm, ones_hbm, zeros_hbm, deg_hbm,
              deg_s_sh, deg_d_sh,
              src_v, dst_v, ones_v, buf_s, buf_d,
              sem_in, sem_s):
    cid = lax.axis_index("c")
    sid = lax.axis_index("s")
    ebase = (cid * NUM_TILES + sid) * EPW
    nbase = sid * NPT

    p0 = [
        pltpu.async_copy(zeros_hbm, deg_s_sh.at[pl.ds(nbase, NPT)], sem_in),
        pltpu.async_copy(zeros_hbm, deg_d_sh.at[pl.ds(nbase, NPT)], sem_in),
        pltpu.async_copy(ei_hbm.at[pl.ds(ebase, EPW)],
                         src_v.at[pl.ds(0, EPW)], sem_in),
        pltpu.async_copy(ei_hbm.at[pl.ds(E + ebase, EPW)],
                         dst_v.at[pl.ds(0, EPW)], sem_in),
        pltpu.async_copy(ones_hbm, ones_v, sem_in),
    ]
    for d in p0:
        d.wait()
    plsc.subcore_barrier()

    for g in range(0, len(_WCHUNKS), _K):
        descs = []
        for off, ln in _WCHUNKS[g:g + _K]:
            descs.append(pltpu.async_copy(
                ones_v.at[pl.ds(0, ln)],
                deg_s_sh.at[src_v.at[pl.ds(off, ln)]], sem_s, add=True))
            descs.append(pltpu.async_copy(
                ones_v.at[pl.ds(0, ln)],
                deg_d_sh.at[dst_v.at[pl.ds(off, ln)]], sem_s, add=True))
        for d in descs:
            d.wait()
    plsc.subcore_barrier()

    o_s = (2 * cid) * N_PAD + nbase
    o_d = (2 * cid + 1) * N_PAD + nbase
    a = pltpu.async_copy(deg_s_sh.at[pl.ds(nbase, NPT)],
                         deg_hbm.at[pl.ds(o_s, NPT)], sem_in)
    b = pltpu.async_copy(deg_d_sh.at[pl.ds(nbase, NPT)],
                         deg_hbm.at[pl.ds(o_d, NPT)], sem_in)
    a.wait()
    b.wait()


def _scb_body(ei_hbm, t_hbm, deg_hbm, zeros_hbm, agg_hbm,
              s_sh, agg_sh,
              src_v, dst_v, vals_v, buf_t, buf_a, buf_b, buf_c,
              sem_in, sem_g, sem_s):
    cid = lax.axis_index("c")
    sid = lax.axis_index("s")
    ebase = (cid * NUM_TILES + sid) * EPW
    nbase = sid * NPT

    p0 = [
        pltpu.async_copy(zeros_hbm, agg_sh.at[pl.ds(nbase, NPT)], sem_in),
        pltpu.async_copy(ei_hbm.at[pl.ds(ebase, EPW)],
                         src_v.at[pl.ds(0, EPW)], sem_in),
        pltpu.async_copy(ei_hbm.at[pl.ds(E + ebase, EPW)],
                         dst_v.at[pl.ds(0, EPW)], sem_in),
        pltpu.async_copy(t_hbm.at[pl.ds(nbase, NPT)], buf_t, sem_in),
        pltpu.async_copy(deg_hbm.at[pl.ds(nbase, NPT)], buf_a, sem_in),
        pltpu.async_copy(deg_hbm.at[pl.ds(2 * N_PAD + nbase, NPT)],
                         buf_b, sem_in),
    ]
    for d in p0:
        d.wait()

    for i in range(NPT // 16):
        dsum = buf_a[pl.ds(i * 16, 16)] + buf_b[pl.ds(i * 16, 16)]
        dmax = jnp.maximum(dsum, jnp.float32(1.0))
        buf_c[pl.ds(i * 16, 16)] = buf_t[pl.ds(i * 16, 16)] * _rsqrt16(dmax)
    pltpu.sync_copy(buf_c, s_sh.at[pl.ds(nbase, NPT)])
    plsc.subcore_barrier()

    pend = []
    for g in range(0, len(_WCHUNKS), _K):
        gd = [pltpu.async_copy(s_sh.at[src_v.at[pl.ds(off, ln)]],
                               vals_v.at[pl.ds(off, ln)], sem_g)
              for off, ln in _WCHUNKS[g:g + _K]]
        for d in gd:
            d.wait()
        for d in pend:
            d.wait()
        pend = [pltpu.async_copy(vals_v.at[pl.ds(off, ln)],
                                 agg_sh.at[dst_v.at[pl.ds(off, ln)]],
                                 sem_s, add=True)
                for off, ln in _WCHUNKS[g:g + _K]]
    for d in pend:
        d.wait()
    plsc.subcore_barrier()

    pltpu.sync_copy(agg_sh.at[pl.ds(nbase, NPT)],
                    agg_hbm.at[pl.ds(cid * N_PAD + nbase, NPT)])


_sc_mesh = plsc.VectorSubcoreMesh(core_axis_name="c", subcore_axis_name="s")

_sca_call = pl.kernel(
    _sca_body,
    out_type=jax.ShapeDtypeStruct((4 * N_PAD,), jnp.float32),
    mesh=_sc_mesh,
    compiler_params=pltpu.CompilerParams(needs_layout_passes=False),
    scratch_types=[
        pltpu.VMEM_SHARED((N_PAD,), jnp.float32),
        pltpu.VMEM_SHARED((N_PAD,), jnp.float32),
        pltpu.VMEM((EPW,), jnp.int32),
        pltpu.VMEM((EPW,), jnp.int32),
        pltpu.VMEM((128,), jnp.float32),
        pltpu.VMEM((NPT,), jnp.float32),
        pltpu.VMEM((NPT,), jnp.float32),
        pltpu.SemaphoreType.DMA,
        pltpu.SemaphoreType.DMA,
    ],
)

_scb_call = pl.kernel(
    _scb_body,
    out_type=jax.ShapeDtypeStruct((2 * N_PAD,), jnp.float32),
    mesh=_sc_mesh,
    compiler_params=pltpu.CompilerParams(needs_layout_passes=False),
    scratch_types=[
        pltpu.VMEM_SHARED((N_PAD,), jnp.float32),
        pltpu.VMEM_SHARED((N_PAD,), jnp.float32),
        pltpu.VMEM((EPW,), jnp.int32),
        pltpu.VMEM((EPW,), jnp.int32),
        pltpu.VMEM((EPW,), jnp.float32),
        pltpu.VMEM((NPT,), jnp.float32),
        pltpu.VMEM((NPT,), jnp.float32),
        pltpu.VMEM((NPT,), jnp.float32),
        pltpu.VMEM((NPT,), jnp.float32),
        pltpu.SemaphoreType.DMA,
        pltpu.SemaphoreType.DMA,
        pltpu.SemaphoreType.DMA,
    ],
)


@jax.jit
def kernel(x, edge_index, W, b, Wc, bc):
    t1d, c2d = _tc_call(x, W, Wc, b, bc)
    ei = jnp.reshape(edge_index, (2 * E,))
    ones1 = jnp.ones((128,), jnp.float32)
    zeros1 = jnp.zeros((NPT,), jnp.float32)
    deg = _sca_call(ei, ones1, zeros1)
    agg = _scb_call(ei, t1d, deg, zeros1)
    y_pad = _tc2_call(deg, agg, c2d)
    return y_pad[:N].reshape(N, 1)
